# baseline (speedup 1.0000x reference)
import functools

import jax
import jax.numpy as jnp
from jax import lax
from jax.experimental import pallas as pl
from jax.experimental.pallas import tpu as pltpu
from jax.experimental.pallas import tpu_sc as plsc

_CHUNK = 128


def _make_gather(num_rows, emb_dim, batch):
    info = plsc.get_sparse_core_info()
    num_workers = info.num_cores * info.num_subcores
    rows_per_worker = batch // num_workers
    chunks = rows_per_worker // _CHUNK
    assert batch == num_workers * chunks * _CHUNK
    mesh = plsc.VectorSubcoreMesh(core_axis_name="c", subcore_axis_name="s")

    @functools.partial(
        pl.kernel,
        out_type=jax.ShapeDtypeStruct((batch, emb_dim), jnp.float32),
        mesh=mesh,
        scratch_types=[
            pltpu.VMEM((chunks, _CHUNK), jnp.int32),
            pltpu.VMEM((rows_per_worker, emb_dim), jnp.float32),
            pltpu.SemaphoreType.DMA,
        ],
        compiler_params=pltpu.CompilerParams(use_tc_tiling_on_sc=False),
    )
    def gather_kernel(table_hbm, idx_hbm, out_hbm, idx_v, rows_v, sem):
        wid = lax.axis_index("s") * info.num_cores + lax.axis_index("c")
        pltpu.sync_copy(idx_hbm.at[wid], idx_v)
        copies = [
            pltpu.async_copy(
                table_hbm.at[idx_v.at[j]],
                rows_v.at[pl.ds(j * _CHUNK, _CHUNK)],
                sem,
            )
            for j in range(chunks)
        ]
        for c in copies:
            c.wait()
        base = wid * rows_per_worker
        pltpu.sync_copy(rows_v, out_hbm.at[pl.ds(base, rows_per_worker)])

    return gather_kernel


def kernel(region_id, embedding):
    idx = region_id.astype(jnp.int32)
    batch = idx.shape[0]
    num_rows, emb_dim = embedding.shape
    info = plsc.get_sparse_core_info()
    num_workers = info.num_cores * info.num_subcores
    chunks = batch // (num_workers * _CHUNK)
    idx3 = idx.reshape(num_workers, chunks, _CHUNK)
    return _make_gather(num_rows, emb_dim, batch)(embedding, idx3)


# speedup vs baseline: 2.1786x; 2.1786x over previous
import functools

import jax
import jax.numpy as jnp
from jax import lax
from jax.experimental import pallas as pl
from jax.experimental.pallas import tpu as pltpu
from jax.experimental.pallas import tpu_sc as plsc

_D = 64
_N = 1000000
_B = 16384
_CW = 256
_NFULL = _N // _CW
_TAIL = _N - _NFULL * _CW
_PAD_ROWS = 32


def _make_gather():
    info = plsc.get_sparse_core_info()
    nw = info.num_cores * info.num_subcores
    per = _NFULL // nw
    rem = _NFULL % nw
    mesh = plsc.VectorSubcoreMesh(core_axis_name="c", subcore_axis_name="s")

    @functools.partial(
        pl.kernel,
        out_type=jax.ShapeDtypeStruct((_B + _PAD_ROWS, 128), jnp.float32),
        mesh=mesh,
        scratch_types=[
            pltpu.VMEM((_B,), jnp.int32),
            pltpu.VMEM((_B,), jnp.int32),
            pltpu.VMEM((_B,), jnp.int32),
            pltpu.VMEM((_B,), jnp.int32),
            pltpu.VMEM((_B,), jnp.int32),
            pltpu.VMEM((_D, _CW), jnp.float32),
            pltpu.VMEM((_D, _CW), jnp.float32),
            pltpu.VMEM((16, 128), jnp.float32),
            pltpu.SemaphoreType.DMA,
            pltpu.SemaphoreType.DMA,
        ],
        compiler_params=pltpu.CompilerParams(needs_layout_passes=False),
    )
    def gather_kernel(table_hbm, tail_hbm, idx_hbm, out_hbm, idxv, pidx, pb,
                      cpidx, cpb, slab0, slab1, rows, sem_s, sem_w):
        wid = lax.axis_index("s") * info.num_cores + lax.axis_index("c")
        nc = per + jnp.where(wid < rem, 1, 0)
        c0 = wid * per + jnp.minimum(wid, rem)
        nct = nc + jnp.where(wid == nw - 1, 1, 0)
        lo = c0 * _CW
        hi = jnp.where(wid == nw - 1, _N, (c0 + nc) * _CW)
        dummy = _B + wid
        lanes = lax.iota(jnp.int32, 16)

        pltpu.sync_copy(idx_hbm, idxv)

        def scan_body(q, cw):
            v = idxv[pl.ds(q * 16, 16)]
            m = (v >= lo) & (v < hi)
            mi = m.astype(jnp.int32)
            pos = jnp.clip(cw + plsc.cumsum(mi) - 1, 0, _B - 1)
            plsc.store_scatter(pidx, [pos], v, mask=m)
            plsc.store_scatter(pb, [pos], lanes + q * 16, mask=m)
            return cw + jnp.sum(mi)

        cw = lax.fori_loop(0, _B // 16, scan_body, jnp.int32(0))

        def fire(k, slab):
            is_tail = (wid == nw - 1) & (k == nc)

            @pl.when((k < nct) & ~is_tail)
            def _():
                cbase = pl.multiple_of((c0 + k) * _CW, _CW)
                pltpu.async_copy(
                    table_hbm.at[:, pl.ds(cbase, _CW)], slab, sem_s)

            @pl.when(is_tail)
            def _():
                pltpu.async_copy(
                    tail_hbm, slab.at[:, pl.ds(0, 128)], sem_s)

        def wait_slab(k):
            is_tail = (wid == nw - 1) & (k == nc)

            @pl.when((k < nct) & ~is_tail)
            def _():
                pltpu.make_async_copy(
                    table_hbm.at[:, pl.ds(0, _CW)], slab0, sem_s).wait()

            @pl.when(is_tail)
            def _():
                pltpu.make_async_copy(
                    tail_hbm, slab0.at[:, pl.ds(0, 128)], sem_s).wait()

        def wait_scat():
            pltpu.make_async_copy(
                rows, out_hbm.at[pl.ds(0, 16)], sem_w).wait()

        def proc(k, slab, gcount):
            is_tail = (wid == nw - 1) & (k == nc)
            cbase = (c0 + k) * _CW
            col_base = jnp.where(is_tail, _N - 128, cbase)
            mlo = jnp.where(is_tail, _N - _TAIL, cbase)
            mhi = jnp.where(is_tail, _N, cbase + _CW)
            wmax = jnp.where(is_tail, 127, _CW - 1)
            valid_chunk = k < nct

            def rs(q, cc):
                v = pidx[pl.ds(q * 16, 16)]
                bv = pb[pl.ds(q * 16, 16)]
                m = ((lanes < (cw - q * 16)) & (v >= mlo)
                     & (v < mhi) & valid_chunk)
                mi = m.astype(jnp.int32)
                pos = jnp.clip(cc + plsc.cumsum(mi) - 1, 0, _B - 1)
                plsc.store_scatter(cpidx, [pos], v, mask=m)
                plsc.store_scatter(cpb, [pos], bv, mask=m)
                return cc + jnp.sum(mi)

            cc = lax.fori_loop(0, (cw + 15) // 16, rs, jnp.int32(0))

            def eg(g, gcnt):
                iv = cpidx[pl.ds(g * 16, 16)]
                bv = cpb[pl.ds(g * 16, 16)]
                lanem = lanes < (cc - g * 16)
                cols = jnp.clip(iv - col_base, 0, wmax)
                bsafe = jnp.where(lanem, bv, dummy)

                @pl.when(gcnt > 0)
                def _():
                    wait_scat()

                for j in range(16):
                    csp = jnp.full((16,), cols[j], dtype=jnp.int32)
                    for t in range(4):
                        vals = plsc.load_gather(
                            slab, [lanes + t * 16, csp])
                        rows[j, pl.ds(t * 16, 16)] = vals
                pltpu.async_copy(rows, out_hbm.at[bsafe], sem_w)
                return gcnt + 1

            return lax.fori_loop(0, (cc + 15) // 16, eg, gcount)

        fire(jnp.int32(0), slab0)

        def chunk_body(k2, gcount):
            a = k2 * 2
            b = a + 1
            wait_slab(a)
            fire(b, slab1)
            gcount = proc(a, slab0, gcount)
            wait_slab(b)
            fire(b + 1, slab0)
            gcount = proc(b, slab1, gcount)
            return gcount

        nk2 = (per + 1 + 1 + 1) // 2
        gcount = lax.fori_loop(0, nk2, chunk_body, jnp.int32(0))

        @pl.when(gcount > 0)
        def _():
            wait_scat()

    return gather_kernel


def kernel(region_id, embedding):
    idx = region_id.astype(jnp.int32)
    tail_t = embedding[_N - 128:, :].T
    out2 = _make_gather()(embedding.T, tail_t, idx)
    return out2[:_B, :_D]


# speedup vs baseline: 3.2292x; 1.4822x over previous
import functools

import jax
import jax.numpy as jnp
from jax import lax
from jax.experimental import pallas as pl
from jax.experimental.pallas import tpu as pltpu
from jax.experimental.pallas import tpu_sc as plsc

_D = 64
_N = 1000000
_B = 16384
_CW = 512
_NFULL = _N // _CW
_TAIL = _N - _NFULL * _CW
_PAD_ROWS = 32


def _make_gather():
    info = plsc.get_sparse_core_info()
    nw = info.num_cores * info.num_subcores
    per = _NFULL // nw
    rem = _NFULL % nw
    mesh = plsc.VectorSubcoreMesh(core_axis_name="c", subcore_axis_name="s")

    @functools.partial(
        pl.kernel,
        out_type=jax.ShapeDtypeStruct((_B + _PAD_ROWS, 128), jnp.float32),
        mesh=mesh,
        scratch_types=[
            pltpu.VMEM((_B,), jnp.int32),
            pltpu.VMEM((_B,), jnp.int32),
            pltpu.VMEM((_B,), jnp.int32),
            pltpu.VMEM((_D, _CW), jnp.float32),
            pltpu.VMEM((_D, _CW), jnp.float32),
            pltpu.VMEM((16, 128), jnp.float32),
            pltpu.SemaphoreType.DMA,
            pltpu.SemaphoreType.DMA,
        ],
        compiler_params=pltpu.CompilerParams(needs_layout_passes=False),
    )
    def gather_kernel(table_hbm, tail_hbm, idx_hbm, out_hbm, pidx, pb,
                      cpos, slab0, slab1, rows, sem_s, sem_w):
        wid = lax.axis_index("s") * info.num_cores + lax.axis_index("c")
        nc = per + jnp.where(wid < rem, 1, 0)
        c0 = wid * per + jnp.minimum(wid, rem)
        nct = nc + jnp.where(wid == nw - 1, 1, 0)
        lo = c0 * _CW
        hi = jnp.where(wid == nw - 1, _N, (c0 + nc) * _CW)
        dummy = _B + wid
        lanes = lax.iota(jnp.int32, 16)

        pltpu.sync_copy(idx_hbm, pidx)

        def scan_body(q, cw):
            v = pidx[pl.ds(q * 16, 16)]
            m = (v >= lo) & (v < hi)
            cs = plsc.cumsum(m.astype(jnp.int32))
            pos = jnp.clip(cw + cs - 1, 0, _B - 1)
            plsc.store_scatter(pidx, [pos], v, mask=m)
            plsc.store_scatter(pb, [pos], lanes + q * 16, mask=m)
            return cw + cs[15]

        cw = lax.fori_loop(0, _B // 16, scan_body, jnp.int32(0))

        def fire(k, slab):
            is_tail = (wid == nw - 1) & (k == nc)

            @pl.when((k < nct) & ~is_tail)
            def _():
                cbase = pl.multiple_of((c0 + k) * _CW, _CW)
                pltpu.async_copy(
                    table_hbm.at[:, pl.ds(cbase, _CW)], slab, sem_s)

            @pl.when(is_tail)
            def _():
                pltpu.async_copy(
                    tail_hbm, slab.at[:, pl.ds(0, 128)], sem_s)

        def wait_slab(k):
            is_tail = (wid == nw - 1) & (k == nc)

            @pl.when((k < nct) & ~is_tail)
            def _():
                pltpu.make_async_copy(
                    table_hbm.at[:, pl.ds(0, _CW)], slab0, sem_s).wait()

            @pl.when(is_tail)
            def _():
                pltpu.make_async_copy(
                    tail_hbm, slab0.at[:, pl.ds(0, 128)], sem_s).wait()

        def wait_scat():
            pltpu.make_async_copy(
                rows, out_hbm.at[pl.ds(0, 16)], sem_w).wait()

        def proc(k, slab, gcount):
            is_tail = (wid == nw - 1) & (k == nc)
            cbase = (c0 + k) * _CW
            col_base = jnp.where(is_tail, _N - 128, cbase)
            mlo = jnp.where(is_tail, _N - _TAIL, cbase)
            mhi = jnp.where(is_tail, _N, cbase + _CW)
            wmax = jnp.where(is_tail, 127, _CW - 1)
            valid_chunk = k < nct

            def rs(q, cc):
                v = pidx[pl.ds(q * 16, 16)]
                m = ((lanes < (cw - q * 16)) & (v >= mlo)
                     & (v < mhi) & valid_chunk)
                cs = plsc.cumsum(m.astype(jnp.int32))
                pos = jnp.clip(cc + cs - 1, 0, _B - 1)
                plsc.store_scatter(cpos, [pos], lanes + q * 16, mask=m)
                return cc + cs[15]

            cc = lax.fori_loop(0, (cw + 15) // 16, rs, jnp.int32(0))

            def eg(g, gcnt):
                slot = jnp.clip(cpos[pl.ds(g * 16, 16)], 0, _B - 1)
                iv = plsc.load_gather(pidx, [slot])
                bv = plsc.load_gather(pb, [slot])
                lanem = lanes < (cc - g * 16)
                cols = jnp.clip(iv - col_base, 0, wmax)
                bsafe = jnp.where(lanem, bv, dummy)

                @pl.when(gcnt > 0)
                def _():
                    wait_scat()

                for j in range(16):
                    csp = jnp.full((16,), cols[j], dtype=jnp.int32)
                    for t in range(4):
                        vals = plsc.load_gather(
                            slab, [lanes + t * 16, csp])
                        rows[j, pl.ds(t * 16, 16)] = vals
                pltpu.async_copy(rows, out_hbm.at[bsafe], sem_w)
                return gcnt + 1

            return lax.fori_loop(0, (cc + 15) // 16, eg, gcount)

        fire(jnp.int32(0), slab0)

        def chunk_body(k2, gcount):
            a = k2 * 2
            b = a + 1
            wait_slab(a)
            fire(b, slab1)
            gcount = proc(a, slab0, gcount)
            wait_slab(b)
            fire(b + 1, slab0)
            gcount = proc(b, slab1, gcount)
            return gcount

        nk2 = (per + 3) // 2
        gcount = lax.fori_loop(0, nk2, chunk_body, jnp.int32(0))

        @pl.when(gcount > 0)
        def _():
            wait_scat()

    return gather_kernel


def kernel(region_id, embedding):
    idx = region_id.astype(jnp.int32)
    tail_t = embedding[_N - 128:, :].T
    out2 = _make_gather()(embedding.T, tail_t, idx)
    return out2[:_B, :_D]


# speedup vs baseline: 3.3243x; 1.0294x over previous
import functools

import jax
import jax.numpy as jnp
from jax import lax
from jax.experimental import pallas as pl
from jax.experimental.pallas import tpu as pltpu
from jax.experimental.pallas import tpu_sc as plsc

_D = 64
_N = 1000000
_B = 16384
_CW = 512
_NFULL = _N // _CW
_TAIL = _N - _NFULL * _CW
_PAD_ROWS = 32


def _make_gather():
    info = plsc.get_sparse_core_info()
    nw = info.num_cores * info.num_subcores
    per = _NFULL // nw
    rem = _NFULL % nw
    mesh = plsc.VectorSubcoreMesh(core_axis_name="c", subcore_axis_name="s")

    @functools.partial(
        pl.kernel,
        out_type=jax.ShapeDtypeStruct((_B + _PAD_ROWS, 128), jnp.float32),
        mesh=mesh,
        scratch_types=[
            pltpu.VMEM((_B,), jnp.int32),
            pltpu.VMEM((_B,), jnp.int32),
            pltpu.VMEM((_B,), jnp.int32),
            pltpu.VMEM((_D, _CW), jnp.float32),
            pltpu.VMEM((_D, _CW), jnp.float32),
            pltpu.VMEM((16, 128), jnp.float32),
            pltpu.SemaphoreType.DMA,
            pltpu.SemaphoreType.DMA,
        ],
        compiler_params=pltpu.CompilerParams(needs_layout_passes=False),
    )
    def gather_kernel(table_hbm, tail_hbm, idx_hbm, out_hbm, pidx, pb,
                      cpos, slab0, slab1, rows, sem_s, sem_w):
        wid = lax.axis_index("s") * info.num_cores + lax.axis_index("c")
        nc = per + jnp.where(wid < rem, 1, 0)
        c0 = wid * per + jnp.minimum(wid, rem)
        nct = nc + jnp.where(wid == nw - 1, 1, 0)
        lo = c0 * _CW
        hi = jnp.where(wid == nw - 1, _N, (c0 + nc) * _CW)
        dummy = _B + wid
        lanes = lax.iota(jnp.int32, 16)

        pltpu.sync_copy(idx_hbm, pidx)

        def fire(k, slab):
            is_tail = (wid == nw - 1) & (k == nc)

            @pl.when((k < nct) & ~is_tail)
            def _():
                cbase = pl.multiple_of((c0 + k) * _CW, _CW)
                pltpu.async_copy(
                    table_hbm.at[:, pl.ds(cbase, _CW)], slab, sem_s)

            @pl.when(is_tail)
            def _():
                pltpu.async_copy(
                    tail_hbm, slab.at[:, pl.ds(0, 128)], sem_s)

        def wait_slab(k):
            is_tail = (wid == nw - 1) & (k == nc)

            @pl.when((k < nct) & ~is_tail)
            def _():
                pltpu.make_async_copy(
                    table_hbm.at[:, pl.ds(0, _CW)], slab0, sem_s).wait()

            @pl.when(is_tail)
            def _():
                pltpu.make_async_copy(
                    tail_hbm, slab0.at[:, pl.ds(0, 128)], sem_s).wait()

        def wait_scat():
            pltpu.make_async_copy(
                rows, out_hbm.at[pl.ds(0, 16)], sem_w).wait()

        fire(jnp.int32(0), slab0)
        fire(jnp.int32(1), slab1)

        def scan_body(q, cw):
            v = pidx[pl.ds(q * 16, 16)]
            m = (v >= lo) & (v < hi)
            cs = plsc.cumsum(m.astype(jnp.int32))
            pos = jnp.clip(cw + cs - 1, 0, _B - 1)
            plsc.store_scatter(pidx, [pos], v, mask=m)
            plsc.store_scatter(pb, [pos], lanes + q * 16, mask=m)
            return cw + cs[15]

        cw = lax.fori_loop(0, _B // 16, scan_body, jnp.int32(0))

        def proc(k, slab, gcount):
            is_tail = (wid == nw - 1) & (k == nc)
            cbase = (c0 + k) * _CW
            col_base = jnp.where(is_tail, _N - 128, cbase)
            mlo = jnp.where(is_tail, _N - _TAIL, cbase)
            mhi = jnp.where(is_tail, _N, cbase + _CW)
            wmax = jnp.where(is_tail, 127, _CW - 1)
            valid_chunk = k < nct

            def rs(q, cc):
                v = pidx[pl.ds(q * 16, 16)]
                m = ((lanes < (cw - q * 16)) & (v >= mlo)
                     & (v < mhi) & valid_chunk)
                cs = plsc.cumsum(m.astype(jnp.int32))
                pos = jnp.clip(cc + cs - 1, 0, _B - 1)
                plsc.store_scatter(cpos, [pos], lanes + q * 16, mask=m)
                return cc + cs[15]

            cc = lax.fori_loop(0, (cw + 15) // 16, rs, jnp.int32(0))

            def eg(g, gcnt):
                slot = jnp.clip(cpos[pl.ds(g * 16, 16)], 0, _B - 1)
                iv = plsc.load_gather(pidx, [slot])
                bv = plsc.load_gather(pb, [slot])
                lanem = lanes < (cc - g * 16)
                cols = jnp.clip(iv - col_base, 0, wmax)
                bsafe = jnp.where(lanem, bv, dummy)

                @pl.when(gcnt > 0)
                def _():
                    wait_scat()

                for j in range(16):
                    csp = jnp.full((16,), cols[j], dtype=jnp.int32)
                    for t in range(4):
                        vals = plsc.load_gather(
                            slab, [lanes + t * 16, csp])
                        rows[j, pl.ds(t * 16, 16)] = vals
                pltpu.async_copy(rows, out_hbm.at[bsafe], sem_w)
                return gcnt + 1

            return lax.fori_loop(0, (cc + 15) // 16, eg, gcount)

        def chunk_body(k2, gcount):
            a = k2 * 2
            b = a + 1
            wait_slab(a)
            gcount = proc(a, slab0, gcount)
            fire(a + 2, slab0)
            wait_slab(b)
            gcount = proc(b, slab1, gcount)
            fire(b + 2, slab1)
            return gcount

        nk2 = (per + 3) // 2
        gcount = lax.fori_loop(0, nk2, chunk_body, jnp.int32(0))

        @pl.when(gcount > 0)
        def _():
            wait_scat()

    return gather_kernel


def kernel(region_id, embedding):
    idx = region_id.astype(jnp.int32)
    tail_t = embedding[_N - 128:, :].T
    out2 = _make_gather()(embedding.T, tail_t, idx)
    return out2[:_B, :_D]
